# Initial kernel scaffold; baseline (speedup 1.0000x reference)
import jax
import jax.numpy as jnp
from jax.experimental import pallas as pl


def kernel(fitnesses, genes, temperature):
    raise NotImplementedError("write your pallas kernel here")


# speedup vs baseline: 2.0191x; 2.0191x over previous
import functools

import jax
import jax.numpy as jnp
from jax import lax
from jax.experimental import pallas as pl
from jax.experimental.pallas import tpu as pltpu
from jax.experimental.pallas import tpu_sc as plsc

NUM_GENES = 16384
NUM_SELECTED = 1024
TOURNAMENT_SIZE = 32
DIM = 512
NUM_CHILDREN = NUM_GENES - NUM_SELECTED


@functools.cache
def _build_constants():
    kperm, klerp = jax.random.split(jax.random.key(42))
    rp = jax.random.normal(
        kperm, (NUM_CHILDREN, NUM_SELECTED), dtype=jnp.float32
    )
    tourn_ids = jnp.argsort(rp, axis=-1)[:, :TOURNAMENT_SIZE].astype(jnp.int32)
    noise = jax.random.normal(klerp, (NUM_CHILDREN, DIM), dtype=jnp.float32)
    order = jnp.argsort(-tourn_ids, axis=-1).astype(jnp.int32)
    dsort = jnp.take_along_axis(tourn_ids, order, axis=-1)
    d1 = dsort[:, :1]
    d2 = dsort[:, 1:2]
    pi1 = order[:, :1]
    return (
        jax.device_put(dsort),
        jax.device_put(order),
        jax.device_put(d1),
        jax.device_put(d2),
        jax.device_put(pi1),
        jax.device_put(noise),
    )


_ROWS_PER_BLOCK = 512


def _children_body(
    m1_ref, m2_ref, pi1_ref, d_ref, pi_ref, parents_ref, noise_ref, temp_ref,
    out_ref,
):
    b = _ROWS_PER_BLOCK
    big = jnp.int32(TOURNAMENT_SIZE * 2)
    m1 = m1_ref[...]
    m2 = m2_ref[...]
    pi1 = pi1_ref[...]
    d = d_ref[...]
    piv = pi_ref[...]
    c1 = jnp.where(d >= m1, piv, big)
    t1 = jnp.min(c1, axis=1, keepdims=True)
    g2 = jnp.where(t1 == pi1, m2, m1)
    c2 = jnp.where((d >= g2) & (piv != t1), piv, big)
    t2 = jnp.min(c2, axis=1, keepdims=True)

    lanes = lax.broadcasted_iota(jnp.int32, (b, TOURNAMENT_SIZE), 1)
    oh1 = (lanes == t1).astype(jnp.float32)
    oh2 = (lanes == t2).astype(jnp.float32)
    parents = parents_ref[...]
    p1 = jnp.dot(oh1, parents, preferred_element_type=jnp.float32)
    p2 = jnp.dot(oh2, parents, preferred_element_type=jnp.float32)

    temp = temp_ref[0, 0]
    x = noise_ref[...] / temp
    w = 1.0 / (1.0 + jnp.exp(-x))
    ch = p1 + (p2 - p1) * w
    n = jnp.sqrt(jnp.sum(ch * ch, axis=1, keepdims=True))
    out_ref[...] = ch / jnp.maximum(n, 1e-12)


def _children(m1, m2, pi1, dsort, order, parents, noise, temperature):
    grid = NUM_CHILDREN // _ROWS_PER_BLOCK
    b = _ROWS_PER_BLOCK
    return pl.pallas_call(
        _children_body,
        grid=(grid,),
        in_specs=[
            pl.BlockSpec((b, 1), lambda i: (i, 0)),
            pl.BlockSpec((b, 1), lambda i: (i, 0)),
            pl.BlockSpec((b, 1), lambda i: (i, 0)),
            pl.BlockSpec((b, TOURNAMENT_SIZE), lambda i: (i, 0)),
            pl.BlockSpec((b, TOURNAMENT_SIZE), lambda i: (i, 0)),
            pl.BlockSpec((TOURNAMENT_SIZE, DIM), lambda i: (0, 0)),
            pl.BlockSpec((b, DIM), lambda i: (i, 0)),
            pl.BlockSpec(memory_space=pltpu.SMEM),
        ],
        out_specs=pl.BlockSpec((b, DIM), lambda i: (i, 0)),
        out_shape=jax.ShapeDtypeStruct((NUM_CHILDREN, DIM), jnp.float32),
    )(m1, m2, pi1, dsort, order, parents, noise, temperature)


def _make_sc_gather():
    info = plsc.get_sparse_core_info()
    nw = info.num_cores * info.num_subcores
    b_per_w = NUM_SELECTED // nw
    mesh = plsc.VectorSubcoreMesh(core_axis_name="c", subcore_axis_name="s")

    @functools.partial(
        pl.kernel,
        mesh=mesh,
        out_type=jax.ShapeDtypeStruct((NUM_SELECTED, DIM), jnp.float32),
        scratch_types=[
            pltpu.VMEM((b_per_w,), jnp.int32),
            pltpu.VMEM((b_per_w, DIM), jnp.float32),
            pltpu.SemaphoreType.DMA,
        ],
    )
    def gather(genes_hbm, idx_hbm, out_hbm, idx_v, rows_v, sem):
        wid = lax.axis_index("s") * info.num_cores + lax.axis_index("c")
        base = wid * b_per_w
        pltpu.sync_copy(idx_hbm.at[pl.ds(base, b_per_w)], idx_v)
        pltpu.async_copy(genes_hbm.at[idx_v], rows_v, sem).wait()
        pltpu.sync_copy(rows_v, out_hbm.at[pl.ds(base, b_per_w)])

    return gather


_SC_GATHER = _make_sc_gather()


def _sortable_key(f):
    u = lax.bitcast_convert_type(f, jnp.int32)
    return jnp.where(u < 0, u ^ jnp.int32(0x7FFFFFFF), u)


def kernel(fitnesses, genes, temperature):
    dsort, order, d1, d2, pi1, noise = _build_constants()
    keys = _sortable_key(fitnesses)
    ids = lax.iota(jnp.int32, NUM_GENES)
    sorted_keys, sorted_ids = lax.sort((keys, ids), num_keys=1, is_stable=True)

    ids_top = sorted_ids[NUM_CHILDREN:]
    keys_bot = sorted_keys[:NUM_SELECTED]

    idx = lax.iota(jnp.int32, NUM_SELECTED)
    first = jnp.concatenate(
        [jnp.ones((1,), jnp.bool_), keys_bot[1:] != keys_bot[:-1]]
    )
    g = lax.cummax(jnp.where(first, idx, 0))
    m1 = jnp.take(g, d1[:, 0]).reshape(NUM_CHILDREN, 1)
    m2 = jnp.take(g, d2[:, 0]).reshape(NUM_CHILDREN, 1)

    parents = jnp.take(genes, ids_top[:TOURNAMENT_SIZE], axis=0)

    temp = jnp.asarray(temperature, jnp.float32).reshape(1, 1)
    children = _children(m1, m2, pi1, dsort, order, parents, noise, temp)
    selected = _SC_GATHER(genes, ids_top)
    return jnp.concatenate([children, selected], axis=0)


# speedup vs baseline: 2.1029x; 1.0415x over previous
import functools

import jax
import jax.numpy as jnp
from jax import lax
from jax.experimental import pallas as pl
from jax.experimental.pallas import tpu as pltpu
from jax.experimental.pallas import tpu_sc as plsc

NUM_GENES = 16384
NUM_SELECTED = 1024
TOURNAMENT_SIZE = 32
DIM = 512
NUM_CHILDREN = NUM_GENES - NUM_SELECTED


@functools.cache
def _build_constants():
    kperm, klerp = jax.random.split(jax.random.key(42))
    rp = jax.random.normal(
        kperm, (NUM_CHILDREN, NUM_SELECTED), dtype=jnp.float32
    )
    tourn_ids = jnp.argsort(rp, axis=-1)[:, :TOURNAMENT_SIZE].astype(jnp.int32)
    noise = jax.random.normal(klerp, (NUM_CHILDREN, DIM), dtype=jnp.float32)
    order = jnp.argsort(-tourn_ids, axis=-1).astype(jnp.int32)
    dsort = jnp.take_along_axis(tourn_ids, order, axis=-1)
    d1 = dsort[:, :1]
    d2 = dsort[:, 1:2]
    pi1 = order[:, :1]
    return (
        jax.device_put(dsort),
        jax.device_put(order),
        jax.device_put(d1),
        jax.device_put(d2),
        jax.device_put(pi1),
        jax.device_put(noise),
    )


_ROWS_PER_BLOCK = 512


def _children_body(
    g_ref, d1_ref, d2_ref, pi1_ref, d_ref, pi_ref, parents_ref, noise_ref,
    temp_ref, out_ref,
):
    b = _ROWS_PER_BLOCK
    big = jnp.int32(TOURNAMENT_SIZE * 2)
    g = g_ref[...]
    cols = lax.broadcasted_iota(jnp.int32, (1, NUM_SELECTED), 1)
    d1 = d1_ref[...]
    d2 = d2_ref[...]
    m1 = jnp.max(jnp.where(cols == d1, g, -1), axis=1, keepdims=True)
    m2 = jnp.max(jnp.where(cols == d2, g, -1), axis=1, keepdims=True)
    pi1 = pi1_ref[...]
    d = d_ref[...]
    piv = pi_ref[...]
    c1 = jnp.where(d >= m1, piv, big)
    t1 = jnp.min(c1, axis=1, keepdims=True)
    g2 = jnp.where(t1 == pi1, m2, m1)
    c2 = jnp.where((d >= g2) & (piv != t1), piv, big)
    t2 = jnp.min(c2, axis=1, keepdims=True)

    lanes = lax.broadcasted_iota(jnp.int32, (b, TOURNAMENT_SIZE), 1)
    oh1 = (lanes == t1).astype(jnp.float32)
    oh2 = (lanes == t2).astype(jnp.float32)
    parents = parents_ref[...]
    p1 = jnp.dot(oh1, parents, preferred_element_type=jnp.float32,
                 precision=lax.Precision.HIGHEST)
    p2 = jnp.dot(oh2, parents, preferred_element_type=jnp.float32,
                 precision=lax.Precision.HIGHEST)

    temp = temp_ref[0, 0]
    x = noise_ref[...] / temp
    w = 1.0 / (1.0 + jnp.exp(-x))
    ch = p1 + (p2 - p1) * w
    n = jnp.sqrt(jnp.sum(ch * ch, axis=1, keepdims=True))
    out_ref[...] = ch / jnp.maximum(n, 1e-12)


def _children(g, d1, d2, pi1, dsort, order, parents, noise, temperature):
    grid = NUM_CHILDREN // _ROWS_PER_BLOCK
    b = _ROWS_PER_BLOCK
    return pl.pallas_call(
        _children_body,
        grid=(grid,),
        in_specs=[
            pl.BlockSpec((1, NUM_SELECTED), lambda i: (0, 0)),
            pl.BlockSpec((b, 1), lambda i: (i, 0)),
            pl.BlockSpec((b, 1), lambda i: (i, 0)),
            pl.BlockSpec((b, 1), lambda i: (i, 0)),
            pl.BlockSpec((b, TOURNAMENT_SIZE), lambda i: (i, 0)),
            pl.BlockSpec((b, TOURNAMENT_SIZE), lambda i: (i, 0)),
            pl.BlockSpec((TOURNAMENT_SIZE, DIM), lambda i: (0, 0)),
            pl.BlockSpec((b, DIM), lambda i: (i, 0)),
            pl.BlockSpec(memory_space=pltpu.SMEM),
        ],
        out_specs=pl.BlockSpec((b, DIM), lambda i: (i, 0)),
        out_shape=jax.ShapeDtypeStruct((NUM_CHILDREN, DIM), jnp.float32),
    )(g, d1, d2, pi1, dsort, order, parents, noise, temperature)


def _make_sc_gather():
    info = plsc.get_sparse_core_info()
    nw = info.num_cores * info.num_subcores
    b_per_w = NUM_SELECTED // nw
    mesh = plsc.VectorSubcoreMesh(core_axis_name="c", subcore_axis_name="s")

    @functools.partial(
        pl.kernel,
        mesh=mesh,
        out_type=jax.ShapeDtypeStruct((NUM_SELECTED, DIM), jnp.float32),
        scratch_types=[
            pltpu.VMEM((b_per_w,), jnp.int32),
            pltpu.VMEM((b_per_w, DIM), jnp.float32),
            pltpu.SemaphoreType.DMA,
        ],
    )
    def gather(genes_hbm, idx_hbm, out_hbm, idx_v, rows_v, sem):
        wid = lax.axis_index("s") * info.num_cores + lax.axis_index("c")
        base = wid * b_per_w
        pltpu.sync_copy(idx_hbm.at[pl.ds(base, b_per_w)], idx_v)
        pltpu.async_copy(genes_hbm.at[idx_v], rows_v, sem).wait()
        pltpu.sync_copy(rows_v, out_hbm.at[pl.ds(base, b_per_w)])

    return gather


_SC_GATHER = _make_sc_gather()


def _sortable_key(f):
    u = lax.bitcast_convert_type(f, jnp.int32)
    return jnp.where(u < 0, u ^ jnp.int32(0x7FFFFFFF), u)


def kernel(fitnesses, genes, temperature):
    dsort, order, d1, d2, pi1, noise = _build_constants()
    keys = _sortable_key(fitnesses)
    ids = lax.iota(jnp.int32, NUM_GENES)
    sorted_keys, sorted_ids = lax.sort((keys, ids), num_keys=1, is_stable=True)

    ids_top = sorted_ids[NUM_CHILDREN:]
    keys_bot = sorted_keys[:NUM_SELECTED]

    idx = lax.iota(jnp.int32, NUM_SELECTED)
    first = jnp.concatenate(
        [jnp.ones((1,), jnp.bool_), keys_bot[1:] != keys_bot[:-1]]
    )
    g = lax.cummax(jnp.where(first, idx, 0))
    parents = jnp.take(genes, ids_top[:TOURNAMENT_SIZE], axis=0)

    temp = jnp.asarray(temperature, jnp.float32).reshape(1, 1)
    children = _children(
        g.reshape(1, NUM_SELECTED), d1, d2, pi1, dsort, order, parents, noise,
        temp,
    )
    selected = _SC_GATHER(genes, ids_top)
    return jnp.concatenate([children, selected], axis=0)


# speedup vs baseline: 2.1145x; 1.0055x over previous
import functools

import jax
import jax.numpy as jnp
from jax import lax
from jax.experimental import pallas as pl
from jax.experimental.pallas import tpu as pltpu
from jax.experimental.pallas import tpu_sc as plsc

NUM_GENES = 16384
NUM_SELECTED = 1024
TOURNAMENT_SIZE = 32
DIM = 512
NUM_CHILDREN = NUM_GENES - NUM_SELECTED


@functools.cache
def _build_constants():
    kperm, klerp = jax.random.split(jax.random.key(42))
    rp = jax.random.normal(
        kperm, (NUM_CHILDREN, NUM_SELECTED), dtype=jnp.float32
    )
    tourn_ids = jnp.argsort(rp, axis=-1)[:, :TOURNAMENT_SIZE].astype(jnp.int32)
    noise = jax.random.normal(klerp, (NUM_CHILDREN, DIM), dtype=jnp.float32)
    order = jnp.argsort(-tourn_ids, axis=-1).astype(jnp.int32)
    dsort = jnp.take_along_axis(tourn_ids, order, axis=-1)
    d1 = dsort[:, :1]
    d2 = dsort[:, 1:2]
    pi1 = order[:, :1]
    return (
        jax.device_put(dsort),
        jax.device_put(order),
        jax.device_put(d1),
        jax.device_put(d2),
        jax.device_put(pi1),
        jax.device_put(noise),
    )


_ROWS_PER_BLOCK = 512


def _children_body(
    g_ref, d1_ref, d2_ref, pi1_ref, d_ref, pi_ref, parents_ref, noise_ref,
    temp_ref, out_ref,
):
    b = _ROWS_PER_BLOCK
    big = jnp.int32(TOURNAMENT_SIZE * 2)
    g = g_ref[...]
    cols = lax.broadcasted_iota(jnp.int32, (1, NUM_SELECTED), 1)
    d1 = d1_ref[...]
    d2 = d2_ref[...]
    m1 = jnp.max(jnp.where(cols == d1, g, -1), axis=1, keepdims=True)
    m2 = jnp.max(jnp.where(cols == d2, g, -1), axis=1, keepdims=True)
    pi1 = pi1_ref[...]
    d = d_ref[...]
    piv = pi_ref[...]
    c1 = jnp.where(d >= m1, piv, big)
    t1 = jnp.min(c1, axis=1, keepdims=True)
    g2 = jnp.where(t1 == pi1, m2, m1)
    c2 = jnp.where((d >= g2) & (piv != t1), piv, big)
    t2 = jnp.min(c2, axis=1, keepdims=True)

    lanes = lax.broadcasted_iota(jnp.int32, (b, TOURNAMENT_SIZE), 1)
    oh1 = (lanes == t1).astype(jnp.float32)
    oh2 = (lanes == t2).astype(jnp.float32)
    parents = parents_ref[...]
    p1 = jnp.dot(oh1, parents, preferred_element_type=jnp.float32,
                 precision=lax.Precision.HIGHEST)
    p2 = jnp.dot(oh2, parents, preferred_element_type=jnp.float32,
                 precision=lax.Precision.HIGHEST)

    temp = temp_ref[0, 0]
    x = noise_ref[...] / temp
    w = 1.0 / (1.0 + jnp.exp(-x))
    ch = p1 + (p2 - p1) * w
    n = jnp.sqrt(jnp.sum(ch * ch, axis=1, keepdims=True))
    out_ref[...] = ch / jnp.maximum(n, 1e-12)


def _children(g, d1, d2, pi1, dsort, order, parents, noise, temperature):
    grid = NUM_CHILDREN // _ROWS_PER_BLOCK
    b = _ROWS_PER_BLOCK
    return pl.pallas_call(
        _children_body,
        grid=(grid,),
        in_specs=[
            pl.BlockSpec((1, NUM_SELECTED), lambda i: (0, 0)),
            pl.BlockSpec((b, 1), lambda i: (i, 0)),
            pl.BlockSpec((b, 1), lambda i: (i, 0)),
            pl.BlockSpec((b, 1), lambda i: (i, 0)),
            pl.BlockSpec((b, TOURNAMENT_SIZE), lambda i: (i, 0)),
            pl.BlockSpec((b, TOURNAMENT_SIZE), lambda i: (i, 0)),
            pl.BlockSpec((TOURNAMENT_SIZE, DIM), lambda i: (0, 0)),
            pl.BlockSpec((b, DIM), lambda i: (i, 0)),
            pl.BlockSpec(memory_space=pltpu.SMEM),
        ],
        out_specs=pl.BlockSpec((b, DIM), lambda i: (i, 0)),
        out_shape=jax.ShapeDtypeStruct((NUM_CHILDREN, DIM), jnp.float32),
    )(g, d1, d2, pi1, dsort, order, parents, noise, temperature)


def _make_sc_gather():
    info = plsc.get_sparse_core_info()
    nw = info.num_cores * info.num_subcores
    b_per_w = NUM_SELECTED // nw
    mesh = plsc.VectorSubcoreMesh(core_axis_name="c", subcore_axis_name="s")

    @functools.partial(
        pl.kernel,
        mesh=mesh,
        out_type=jax.ShapeDtypeStruct((NUM_SELECTED, DIM), jnp.float32),
        scratch_types=[
            pltpu.VMEM((b_per_w,), jnp.int32),
            pltpu.VMEM((b_per_w, DIM), jnp.float32),
            pltpu.SemaphoreType.DMA,
        ],
    )
    def gather(genes_hbm, idx_hbm, out_hbm, idx_v, rows_v, sem):
        wid = lax.axis_index("s") * info.num_cores + lax.axis_index("c")
        base = wid * b_per_w
        pltpu.sync_copy(idx_hbm.at[pl.ds(base, b_per_w)], idx_v)
        pltpu.async_copy(genes_hbm.at[idx_v], rows_v, sem).wait()
        pltpu.sync_copy(rows_v, out_hbm.at[pl.ds(base, b_per_w)])

    return gather


_SC_GATHER = _make_sc_gather()


def _sortable_key(f):
    u = lax.bitcast_convert_type(f, jnp.int32)
    return jnp.where(u < 0, u ^ jnp.int32(0x7FFFFFFF), u)


def kernel(fitnesses, genes, temperature):
    dsort, order, d1, d2, pi1, noise = _build_constants()
    keys = _sortable_key(fitnesses)
    ids = lax.iota(jnp.int32, NUM_GENES)
    sorted_keys, sorted_ids = lax.sort((keys, ids), num_keys=1, is_stable=True)

    ids_top = sorted_ids[NUM_CHILDREN:]
    keys_bot = sorted_keys[:NUM_SELECTED]

    idx = lax.iota(jnp.int32, NUM_SELECTED)
    first = jnp.concatenate(
        [jnp.ones((1,), jnp.bool_), keys_bot[1:] != keys_bot[:-1]]
    )
    g = lax.cummax(jnp.where(first, idx, 0))

    selected = _SC_GATHER(genes, ids_top)
    parents = selected[:TOURNAMENT_SIZE]

    temp = jnp.asarray(temperature, jnp.float32).reshape(1, 1)
    children = _children(
        g.reshape(1, NUM_SELECTED), d1, d2, pi1, dsort, order, parents, noise,
        temp,
    )
    return jnp.concatenate([children, selected], axis=0)
